# Initial kernel scaffold; baseline (speedup 1.0000x reference)
import functools

import jax
import jax.numpy as jnp
from jax import lax
from jax.experimental import pallas as pl
from jax.experimental.pallas import tpu as pltpu
from jax.experimental.pallas import tpu_sc as plsc


def _sc_mesh():
    return plsc.VectorSubcoreMesh(core_axis_name="c", subcore_axis_name="s")


def _make_sc_deg(E, N, NC, NS):
    NW = NC * NS
    EPW = E // NW

    @functools.partial(
        pl.kernel,
        mesh=_sc_mesh(),
        out_type=jax.ShapeDtypeStruct((NW * N,), jnp.float32),
        scratch_types=[
            pltpu.VMEM((EPW,), jnp.int32),
            pltpu.VMEM((N,), jnp.float32),
        ],
    )
    def deg_kernel(dst_hbm, out_hbm, dst_v, acc_v):
        wid = lax.axis_index("s") * NC + lax.axis_index("c")
        pltpu.sync_copy(dst_hbm.at[pl.ds(wid * EPW, EPW)], dst_v)
        zeros = jnp.zeros((16,), jnp.float32)

        def zbody(i, carry):
            acc_v[pl.ds(i * 16, 16)] = zeros
            return carry

        lax.fori_loop(0, N // 16, zbody, 0, unroll=8)
        ones = jnp.ones((16,), jnp.float32)

        def ebody(i, carry):
            di = dst_v[pl.ds(i * 16, 16)]
            plsc.addupdate_scatter(acc_v, [di], ones)
            return carry

        lax.fori_loop(0, EPW // 16, ebody, 0, unroll=8)
        pltpu.sync_copy(acc_v, out_hbm.at[pl.ds(wid * N, N)])

    return deg_kernel


def _make_sc_spmm(E, N, NC, NS):
    NW = NC * NS
    EPW = E // NW

    @functools.partial(
        pl.kernel,
        mesh=_sc_mesh(),
        out_type=jax.ShapeDtypeStruct((NW * N,), jnp.float32),
        scratch_types=[
            pltpu.VMEM((EPW,), jnp.int32),
            pltpu.VMEM((EPW,), jnp.int32),
            pltpu.VMEM((N,), jnp.float32),
            pltpu.VMEM((N,), jnp.float32),
        ],
    )
    def spmm_kernel(src_hbm, dst_hbm, v_hbm, out_hbm, src_v, dst_v, v_v, acc_v):
        wid = lax.axis_index("s") * NC + lax.axis_index("c")
        pltpu.sync_copy(src_hbm.at[pl.ds(wid * EPW, EPW)], src_v)
        pltpu.sync_copy(dst_hbm.at[pl.ds(wid * EPW, EPW)], dst_v)
        pltpu.sync_copy(v_hbm, v_v)
        zeros = jnp.zeros((16,), jnp.float32)

        def zbody(i, carry):
            acc_v[pl.ds(i * 16, 16)] = zeros
            return carry

        lax.fori_loop(0, N // 16, zbody, 0, unroll=8)

        def ebody(i, carry):
            si = src_v[pl.ds(i * 16, 16)]
            di = dst_v[pl.ds(i * 16, 16)]
            vals = plsc.load_gather(v_v, [si])
            plsc.addupdate_scatter(acc_v, [di], vals)
            return carry

        lax.fori_loop(0, EPW // 16, ebody, 0, unroll=4)
        pltpu.sync_copy(acc_v, out_hbm.at[pl.ds(wid * N, N)])

    return spmm_kernel


def _tc_pre(x, s, Wr, degp):
    N, D = x.shape

    def body(x_ref, s_ref, w_ref, degp_ref, dinv_ref, u0_ref, u1_ref, q2_ref):
        deg = jnp.sum(degp_ref[...], axis=0)
        dinv = jnp.where(deg > 0, lax.rsqrt(jnp.maximum(deg, 1e-12)), 0.0)
        ws = s_ref[...][None, :] * w_ref[...]
        u = lax.dot_general(
            x_ref[...], ws, (((1,), (1,)), ((), ())),
            preferred_element_type=jnp.float32,
            precision=lax.Precision.HIGHEST,
        )
        dinv_ref[...] = dinv
        u0_ref[...] = u[:, 0]
        u1_ref[...] = u[:, 1]
        q2_ref[...] = dinv * u[:, 2]

    f32 = jnp.float32
    return pl.pallas_call(
        body,
        out_shape=[jax.ShapeDtypeStruct((N,), f32)] * 4,
    )(x, s, Wr, degp)


def _tc_mid(y1p, u1, dinv):
    N = u1.shape[0]

    def body(y1p_ref, u1_ref, dinv_ref, g_ref):
        y1 = jnp.sum(y1p_ref[...], axis=0)
        dinv = dinv_ref[...]
        g_ref[...] = dinv * (u1_ref[...] + dinv * y1)

    return pl.pallas_call(
        body,
        out_shape=jax.ShapeDtypeStruct((N,), jnp.float32),
    )(y1p, u1, dinv)


def _tc_post(y2p, u0, dinv, b):
    N = u0.shape[0]

    def body(y2p_ref, u0_ref, dinv_ref, b_ref, out_ref):
        y2 = jnp.sum(y2p_ref[...], axis=0)
        out_ref[...] = u0_ref[...] + dinv_ref[...] * y2 + b_ref[...]

    return pl.pallas_call(
        body,
        out_shape=jax.ShapeDtypeStruct((N,), jnp.float32),
    )(y2p, u0, dinv, b)


def kernel(x, edge_index, s, W, b):
    N, D = x.shape
    E = edge_index.shape[1]
    K = W.shape[0] // D

    info = plsc.get_sparse_core_info()
    NC, NS = info.num_cores, info.num_subcores
    NW = NC * NS

    src = edge_index[0]
    dst = edge_index[1]
    Wr = W[:, 0].reshape(K, D)

    deg_k = _make_sc_deg(E, N, NC, NS)
    spmm_k = _make_sc_spmm(E, N, NC, NS)

    degp = deg_k(dst).reshape(NW, N)
    dinv, u0, u1, q2 = _tc_pre(x, s, Wr, degp)
    y1p = spmm_k(src, dst, q2).reshape(NW, N)
    g = _tc_mid(y1p, u1, dinv)
    y2p = spmm_k(src, dst, g).reshape(NW, N)
    out = _tc_post(y2p, u0, dinv, b)
    return out.reshape(N, 1)


# speedup vs baseline: 80.9448x; 80.9448x over previous
import functools

import jax
import jax.numpy as jnp
from jax import lax
from jax.experimental import pallas as pl
from jax.experimental.pallas import tpu as pltpu
from jax.experimental.pallas import tpu_sc as plsc


def _sc_mesh():
    return plsc.VectorSubcoreMesh(core_axis_name="c", subcore_axis_name="s")


def _make_sc_deg(E, N, NC, NS):
    NW = NC * NS
    EPW = E // NW

    @functools.partial(
        pl.kernel,
        mesh=_sc_mesh(),
        out_type=jax.ShapeDtypeStruct((NW * N,), jnp.float32),
        scratch_types=[
            pltpu.VMEM((EPW,), jnp.int32),
            pltpu.VMEM((N,), jnp.float32),
        ],
        compiler_params=pltpu.CompilerParams(needs_layout_passes=False),
    )
    def deg_kernel(dst_hbm, out_hbm, dst_v, acc_v):
        wid = lax.axis_index("s") * NC + lax.axis_index("c")
        pltpu.sync_copy(dst_hbm.at[pl.ds(wid * EPW, EPW)], dst_v)
        zeros = jnp.zeros((16,), jnp.float32)

        def zbody(i, carry):
            acc_v[pl.ds(i * 16, 16)] = zeros
            return carry

        lax.fori_loop(0, N // 16, zbody, 0, unroll=8)
        ones = jnp.ones((16,), jnp.float32)

        def ebody(i, carry):
            di = dst_v[pl.ds(i * 16, 16)]
            plsc.addupdate_scatter(acc_v, [di], ones)
            return carry

        lax.fori_loop(0, EPW // 16, ebody, 0, unroll=8)
        pltpu.sync_copy(acc_v, out_hbm.at[pl.ds(wid * N, N)])

    return deg_kernel


def _make_sc_spmm(E, N, NC, NS):
    NW = NC * NS
    EPW = E // NW

    @functools.partial(
        pl.kernel,
        mesh=_sc_mesh(),
        out_type=jax.ShapeDtypeStruct((NW * N,), jnp.float32),
        scratch_types=[
            pltpu.VMEM((EPW,), jnp.int32),
            pltpu.VMEM((EPW,), jnp.int32),
            pltpu.VMEM((N,), jnp.float32),
            pltpu.VMEM((N,), jnp.float32),
        ],
        compiler_params=pltpu.CompilerParams(needs_layout_passes=False),
    )
    def spmm_kernel(src_hbm, dst_hbm, v_hbm, out_hbm, src_v, dst_v, v_v, acc_v):
        wid = lax.axis_index("s") * NC + lax.axis_index("c")
        pltpu.sync_copy(src_hbm.at[pl.ds(wid * EPW, EPW)], src_v)
        pltpu.sync_copy(dst_hbm.at[pl.ds(wid * EPW, EPW)], dst_v)
        pltpu.sync_copy(v_hbm, v_v)
        zeros = jnp.zeros((16,), jnp.float32)

        def zbody(i, carry):
            acc_v[pl.ds(i * 16, 16)] = zeros
            return carry

        lax.fori_loop(0, N // 16, zbody, 0, unroll=8)

        def ebody(i, carry):
            si = src_v[pl.ds(i * 16, 16)]
            di = dst_v[pl.ds(i * 16, 16)]
            vals = plsc.load_gather(v_v, [si])
            plsc.addupdate_scatter(acc_v, [di], vals)
            return carry

        lax.fori_loop(0, EPW // 16, ebody, 0, unroll=4)
        pltpu.sync_copy(acc_v, out_hbm.at[pl.ds(wid * N, N)])

    return spmm_kernel


def _tc_pre(x, s, Wr, degp):
    N, D = x.shape

    def body(x_ref, s_ref, w_ref, degp_ref, dinv_ref, u0_ref, u1_ref, q2_ref):
        deg = jnp.sum(degp_ref[...], axis=0)
        dinv = jnp.where(deg > 0, lax.rsqrt(jnp.maximum(deg, 1e-12)), 0.0)
        ws = s_ref[...][None, :] * w_ref[...]
        u = lax.dot_general(
            x_ref[...], ws, (((1,), (1,)), ((), ())),
            preferred_element_type=jnp.float32,
            precision=lax.Precision.HIGHEST,
        )
        dinv_ref[...] = dinv
        u0_ref[...] = u[:, 0]
        u1_ref[...] = u[:, 1]
        q2_ref[...] = dinv * u[:, 2]

    f32 = jnp.float32
    return pl.pallas_call(
        body,
        out_shape=[jax.ShapeDtypeStruct((N,), f32)] * 4,
    )(x, s, Wr, degp)


def _tc_mid(y1p, u1, dinv):
    N = u1.shape[0]

    def body(y1p_ref, u1_ref, dinv_ref, g_ref):
        y1 = jnp.sum(y1p_ref[...], axis=0)
        dinv = dinv_ref[...]
        g_ref[...] = dinv * (u1_ref[...] + dinv * y1)

    return pl.pallas_call(
        body,
        out_shape=jax.ShapeDtypeStruct((N,), jnp.float32),
    )(y1p, u1, dinv)


def _tc_post(y2p, u0, dinv, b):
    N = u0.shape[0]

    def body(y2p_ref, u0_ref, dinv_ref, b_ref, out_ref):
        y2 = jnp.sum(y2p_ref[...], axis=0)
        out_ref[...] = u0_ref[...] + dinv_ref[...] * y2 + b_ref[...]

    return pl.pallas_call(
        body,
        out_shape=jax.ShapeDtypeStruct((N,), jnp.float32),
    )(y2p, u0, dinv, b)


def kernel(x, edge_index, s, W, b):
    N, D = x.shape
    E = edge_index.shape[1]
    K = W.shape[0] // D

    info = plsc.get_sparse_core_info()
    NC, NS = info.num_cores, info.num_subcores
    NW = NC * NS

    src = edge_index[0]
    dst = edge_index[1]
    Wr = W[:, 0].reshape(K, D)

    deg_k = _make_sc_deg(E, N, NC, NS)
    spmm_k = _make_sc_spmm(E, N, NC, NS)

    degp = deg_k(dst).reshape(NW, N)
    dinv, u0, u1, q2 = _tc_pre(x, s, Wr, degp)
    y1p = spmm_k(src, dst, q2).reshape(NW, N)
    g = _tc_mid(y1p, u1, dinv)
    y2p = spmm_k(src, dst, g).reshape(NW, N)
    out = _tc_post(y2p, u0, dinv, b)
    return out.reshape(N, 1)


# speedup vs baseline: 96.1924x; 1.1884x over previous
import functools

import jax
import jax.numpy as jnp
from jax import lax
from jax.experimental import pallas as pl
from jax.experimental.pallas import tpu as pltpu
from jax.experimental.pallas import tpu_sc as plsc


def _sc_mesh():
    return plsc.VectorSubcoreMesh(core_axis_name="c", subcore_axis_name="s")


def _make_sc_deg(E, N, NC, NS):
    NW = NC * NS
    EPW = E // NW

    @functools.partial(
        pl.kernel,
        mesh=_sc_mesh(),
        out_type=jax.ShapeDtypeStruct((NW, N), jnp.float32),
        scratch_types=[
            pltpu.VMEM((EPW,), jnp.int32),
            pltpu.VMEM((N,), jnp.float32),
        ],
        compiler_params=pltpu.CompilerParams(needs_layout_passes=False, use_tc_tiling_on_sc=False),
    )
    def deg_kernel(ei_hbm, out_hbm, dst_v, acc_v):
        wid = lax.axis_index("s") * NC + lax.axis_index("c")
        pltpu.sync_copy(ei_hbm.at[1, pl.ds(wid * EPW, EPW)], dst_v)
        zeros = jnp.zeros((16,), jnp.float32)

        def zbody(i, carry):
            acc_v[pl.ds(i * 16, 16)] = zeros
            return carry

        lax.fori_loop(0, N // 16, zbody, 0, unroll=8)
        ones = jnp.ones((16,), jnp.float32)

        def ebody(i, carry):
            di = dst_v[pl.ds(i * 16, 16)]
            plsc.addupdate_scatter(acc_v, [di], ones)
            return carry

        lax.fori_loop(0, EPW // 16, ebody, 0, unroll=8)
        pltpu.sync_copy(acc_v, out_hbm.at[wid])

    return deg_kernel


def _make_sc_spmm(E, N, NC, NS):
    NW = NC * NS
    EPW = E // NW

    @functools.partial(
        pl.kernel,
        mesh=_sc_mesh(),
        out_type=jax.ShapeDtypeStruct((NW, N), jnp.float32),
        scratch_types=[
            pltpu.VMEM((EPW,), jnp.int32),
            pltpu.VMEM((EPW,), jnp.int32),
            pltpu.VMEM((N,), jnp.float32),
            pltpu.VMEM((N,), jnp.float32),
        ],
        compiler_params=pltpu.CompilerParams(needs_layout_passes=False, use_tc_tiling_on_sc=False),
    )
    def spmm_kernel(ei_hbm, v_hbm, out_hbm, src_v, dst_v, v_v, acc_v):
        wid = lax.axis_index("s") * NC + lax.axis_index("c")
        pltpu.sync_copy(ei_hbm.at[0, pl.ds(wid * EPW, EPW)], src_v)
        pltpu.sync_copy(ei_hbm.at[1, pl.ds(wid * EPW, EPW)], dst_v)
        pltpu.sync_copy(v_hbm, v_v)
        zeros = jnp.zeros((16,), jnp.float32)

        def zbody(i, carry):
            acc_v[pl.ds(i * 16, 16)] = zeros
            return carry

        lax.fori_loop(0, N // 16, zbody, 0, unroll=8)

        def ebody(i, carry):
            si = src_v[pl.ds(i * 16, 16)]
            di = dst_v[pl.ds(i * 16, 16)]
            vals = plsc.load_gather(v_v, [si])
            plsc.addupdate_scatter(acc_v, [di], vals)
            return carry

        lax.fori_loop(0, EPW // 16, ebody, 0, unroll=8)
        pltpu.sync_copy(acc_v, out_hbm.at[wid])

    return spmm_kernel


def _tc_matvec(x, s, Wr):
    N, D = x.shape

    def body(x_ref, s_ref, w_ref, u0_ref, u1_ref, u2_ref):
        ws = s_ref[...][None, :] * w_ref[...]
        u = lax.dot_general(
            x_ref[...], ws, (((1,), (1,)), ((), ())),
            preferred_element_type=jnp.float32,
            precision=lax.Precision.HIGHEST,
        )
        u0_ref[...] = u[:, 0]
        u1_ref[...] = u[:, 1]
        u2_ref[...] = u[:, 2]

    f32 = jnp.float32
    return pl.pallas_call(
        body,
        out_shape=[jax.ShapeDtypeStruct((N,), f32)] * 3,
    )(x, s, Wr)


def _tc_dinv(degp, u2):
    N = u2.shape[0]

    def body(degp_ref, u2_ref, dinv_ref, q2_ref):
        deg = jnp.sum(degp_ref[...], axis=0)
        dinv = jnp.where(deg > 0, lax.rsqrt(jnp.maximum(deg, 1e-12)), 0.0)
        dinv_ref[...] = dinv
        q2_ref[...] = dinv * u2_ref[...]

    f32 = jnp.float32
    return pl.pallas_call(
        body,
        out_shape=[jax.ShapeDtypeStruct((N,), f32)] * 2,
    )(degp, u2)


def _tc_mid(y1p, u1, dinv):
    N = u1.shape[0]

    def body(y1p_ref, u1_ref, dinv_ref, g_ref):
        y1 = jnp.sum(y1p_ref[...], axis=0)
        dinv = dinv_ref[...]
        g_ref[...] = dinv * (u1_ref[...] + dinv * y1)

    return pl.pallas_call(
        body,
        out_shape=jax.ShapeDtypeStruct((N,), jnp.float32),
    )(y1p, u1, dinv)


def _tc_post(y2p, u0, dinv, b):
    N = u0.shape[0]

    def body(y2p_ref, u0_ref, dinv_ref, b_ref, out_ref):
        y2 = jnp.sum(y2p_ref[...], axis=0)
        out_ref[...] = u0_ref[...] + dinv_ref[...] * y2 + b_ref[...]

    return pl.pallas_call(
        body,
        out_shape=jax.ShapeDtypeStruct((N,), jnp.float32),
    )(y2p, u0, dinv, b)


def kernel(x, edge_index, s, W, b):
    N, D = x.shape
    E = edge_index.shape[1]
    K = W.shape[0] // D

    info = plsc.get_sparse_core_info()
    NC, NS = info.num_cores, info.num_subcores

    Wr = W[:, 0].reshape(K, D)

    deg_k = _make_sc_deg(E, N, NC, NS)
    spmm_k = _make_sc_spmm(E, N, NC, NS)

    degp = deg_k(edge_index)
    u0, u1, u2 = _tc_matvec(x, s, Wr)
    dinv, q2 = _tc_dinv(degp, u2)
    y1p = spmm_k(edge_index, q2)
    g = _tc_mid(y1p, u1, dinv)
    y2p = spmm_k(edge_index, g)
    out = _tc_post(y2p, u0, dinv, b)
    return out.reshape(N, 1)
